# baseline (speedup 1.0000x reference)
import functools

import jax
import jax.numpy as jnp
from jax import lax
from jax.experimental import pallas as pl
from jax.experimental.pallas import tpu as pltpu
from jax.experimental.pallas import tpu_sc as plsc

_N = 10000
_E = 320000
_NC = 2
_NS = 16
_NL = 16
_K = 80
_EP = _E


def _make_spmm(D, d_scale=None):
    if d_scale is None:
        d_scale = D
    K = _K
    ept = _EP // (_NC * _NS)
    nchunk = ept // K
    NB = 4
    NBI = 8
    NU = 8
    rpt = 624
    rem = _N - _NS * rpt
    mesh = plsc.VectorSubcoreMesh(core_axis_name="c", subcore_axis_name="s")

    @functools.partial(
        pl.kernel,
        out_type=jax.ShapeDtypeStruct((_NC, _N, D), jnp.float32),
        mesh=mesh,
        scratch_types=[
            pltpu.VMEM((NBI, K), jnp.int32),
            pltpu.VMEM((NBI, K), jnp.int32),
            pltpu.VMEM((NBI, K), jnp.float32),
            pltpu.VMEM((NB, K, D), jnp.float32),
            pltpu.VMEM_SHARED((_N, D), jnp.float32),
            pltpu.SemaphoreType.DMA,
            pltpu.SemaphoreType.DMA,
            pltpu.SemaphoreType.DMA,
        ],
    )
    def spmm(row_hbm, col_hbm, w_hbm, mat_hbm, out_hbm,
             rowv, colv, wv, gbuf, acc, isem, gsem, ssem):
        c = lax.axis_index("c")
        s = lax.axis_index("s")
        wid = s * _NC + c
        ebase = wid * ept

        def zrow(k, _):
            def zd(j, _):
                gbuf[0, k, pl.ds(j * _NL, _NL)] = jnp.zeros((_NL,), jnp.float32)
                return 0
            return lax.fori_loop(0, D // _NL, zd, 0)
        lax.fori_loop(0, K, zrow, 0)
        rbase = s * rpt
        nfull, tail = divmod(rpt, K)
        for t in range(nfull):
            pltpu.sync_copy(gbuf.at[0], acc.at[pl.ds(rbase + t * K, K)])
        if tail:
            pltpu.sync_copy(gbuf.at[0].at[pl.ds(0, tail)],
                            acc.at[pl.ds(rbase + nfull * K, tail)])

        @pl.when(s == 0)
        def _zero_rem():
            pltpu.sync_copy(gbuf.at[0].at[pl.ds(0, rem)],
                            acc.at[pl.ds(_NS * rpt, rem)])
        plsc.subcore_barrier()

        def istart(t, slot):
            off = ebase + t * K
            pltpu.async_copy(row_hbm.at[pl.ds(off, K)], rowv.at[slot], isem)
            pltpu.async_copy(col_hbm.at[pl.ds(off, K)], colv.at[slot], isem)
            pltpu.async_copy(w_hbm.at[pl.ds(off, K)], wv.at[slot], isem)

        def iwait():
            for r in (rowv, colv, wv):
                pltpu.make_async_copy(row_hbm.at[pl.ds(0, K)], r.at[0], isem).wait()

        def gstart(islot, b):
            pltpu.async_copy(mat_hbm.at[colv.at[islot]], gbuf.at[b], gsem)

        def gwait():
            pltpu.make_async_copy(mat_hbm.at[colv.at[0]], gbuf.at[0], gsem).wait()

        def sstart(islot, b):
            pltpu.async_copy(gbuf.at[b], acc.at[rowv.at[islot]], ssem, add=True)

        def swait():
            pltpu.make_async_copy(gbuf.at[0], acc.at[rowv.at[0]], ssem).wait()

        def scale(islot, b):
            def sg(g, _):
                w16 = wv[islot, pl.ds(g * _NL, _NL)]
                for k in range(_NL):
                    w = w16[k]
                    e = g * _NL + k
                    for j in range(d_scale // _NL):
                        sl = pl.ds(j * _NL, _NL)
                        gbuf[b, e, sl] = gbuf[b, e, sl] * w
                return 0
            lax.fori_loop(0, K // _NL, sg, 0)

        def step(t, j):
            b = j % NB
            gwait()
            scale(j % NBI, b)
            sstart(j % NBI, b)

            @pl.when(t >= 2)
            def _():
                swait()

            @pl.when(t <= nchunk - 3)
            def _():
                iwait()
                gstart((j + 2) % NBI, (b + 2) % NB)

            @pl.when(t <= nchunk - 6)
            def _():
                istart(t + 5, (j + 5) % NBI)

        for t in range(5):
            istart(t, t)
        iwait()
        iwait()
        gstart(0, 0)
        gstart(1, 1)

        def group(g, _):
            for j in range(NU):
                step(g * NU + j, j)
            return 0
        lax.fori_loop(0, nchunk // NU, group, 0)
        for t in range(NU * (nchunk // NU), nchunk):
            step(t, t % NU)
        swait()
        swait()
        plsc.subcore_barrier()

        pltpu.sync_copy(acc.at[pl.ds(rbase, rpt)], out_hbm.at[c, pl.ds(rbase, rpt)])

        @pl.when(s == 0)
        def _write_rem():
            pltpu.sync_copy(acc.at[pl.ds(_NS * rpt, rem)],
                            out_hbm.at[c, pl.ds(_NS * rpt, rem)])

    return spmm


_spmm128 = _make_spmm(128)
_spmm48 = _make_spmm(128, d_scale=48)


def _tc_matmul1(x, W1):
    bn = 1000

    def body(x_ref, w_ref, o_ref):
        o_ref[...] = jnp.dot(x_ref[...], w_ref[...],
                             preferred_element_type=jnp.float32)

    return pl.pallas_call(
        body,
        grid=(_N // bn,),
        in_specs=[pl.BlockSpec((bn, 128), lambda i: (i, 0)),
                  pl.BlockSpec((128, 128), lambda i: (0, 0))],
        out_specs=pl.BlockSpec((bn, 128), lambda i: (i, 0)),
        out_shape=jax.ShapeDtypeStruct((_N, 128), jnp.float32),
    )(x, W1)


def _tc_layer2(p0, p1, b1, W2p):
    bn = 1000

    def body(p0_ref, p1_ref, b_ref, w_ref, o_ref):
        h = jnp.maximum(p0_ref[...] + p1_ref[...] + b_ref[...], 0.0)
        o_ref[...] = jnp.dot(h, w_ref[...], preferred_element_type=jnp.float32)

    return pl.pallas_call(
        body,
        grid=(_N // bn,),
        in_specs=[pl.BlockSpec((bn, 128), lambda i: (i, 0)),
                  pl.BlockSpec((bn, 128), lambda i: (i, 0)),
                  pl.BlockSpec((1, 128), lambda i: (0, 0)),
                  pl.BlockSpec((128, 128), lambda i: (0, 0))],
        out_specs=pl.BlockSpec((bn, 128), lambda i: (i, 0)),
        out_shape=jax.ShapeDtypeStruct((_N, 128), jnp.float32),
    )(p0, p1, b1, W2p)


def _tc_final(q0, q1, b2):
    bn = 1000

    def body(q0_ref, q1_ref, b_ref, o_ref):
        z = (q0_ref[...] + q1_ref[...])[:, :40] + b_ref[...]
        z = z - jnp.max(z, axis=1, keepdims=True)
        o_ref[...] = z - jnp.log(jnp.sum(jnp.exp(z), axis=1, keepdims=True))

    return pl.pallas_call(
        body,
        grid=(_N // bn,),
        in_specs=[pl.BlockSpec((bn, 128), lambda i: (i, 0)),
                  pl.BlockSpec((bn, 128), lambda i: (i, 0)),
                  pl.BlockSpec((1, 40), lambda i: (0, 0))],
        out_specs=pl.BlockSpec((bn, 40), lambda i: (i, 0)),
        out_shape=jax.ShapeDtypeStruct((_N, 40), jnp.float32),
    )(q0, q1, b2)


def kernel(x, edge_index, edge_weight, W1, b1, W2, b2):
    row = edge_index[0].astype(jnp.int32)
    col = edge_index[1].astype(jnp.int32)
    ew = edge_weight.astype(jnp.float32)

    s1 = _tc_matmul1(x, W1)
    p = _spmm128(row, col, ew, s1)
    W2p = jnp.pad(W2, ((0, 0), (0, 128 - W2.shape[1])))
    s2 = _tc_layer2(p[0], p[1], b1.reshape(1, -1), W2p)
    q = _spmm48(row, col, ew, s2)
    return _tc_final(q[0], q[1], b2.reshape(1, -1))


# speedup vs baseline: 1.0036x; 1.0036x over previous
import functools

import jax
import jax.numpy as jnp
from jax import lax
from jax.experimental import pallas as pl
from jax.experimental.pallas import tpu as pltpu
from jax.experimental.pallas import tpu_sc as plsc

_N = 10000
_E = 320000
_NC = 2
_NS = 16
_NL = 16
_K = 80
_EP = _E


def _make_spmm(D, d_scale=None):
    if d_scale is None:
        d_scale = D
    K = _K
    ept = _EP // (_NC * _NS)
    nchunk = ept // K
    NB = 4
    NBI = 8
    NU = 8
    rpt = 624
    rem = _N - _NS * rpt
    mesh = plsc.VectorSubcoreMesh(core_axis_name="c", subcore_axis_name="s")

    @functools.partial(
        pl.kernel,
        out_type=jax.ShapeDtypeStruct((_NC, _N, D), jnp.float32),
        mesh=mesh,
        scratch_types=[
            pltpu.VMEM((NBI, K), jnp.int32),
            pltpu.VMEM((NBI, K), jnp.int32),
            pltpu.VMEM((NBI, K), jnp.float32),
            pltpu.VMEM((NB, K, D), jnp.float32),
            pltpu.VMEM_SHARED((_N, D), jnp.float32),
            pltpu.SemaphoreType.DMA,
            pltpu.SemaphoreType.DMA,
            pltpu.SemaphoreType.DMA,
        ],
    )
    def spmm(row_hbm, col_hbm, w_hbm, mat_hbm, out_hbm,
             rowv, colv, wv, gbuf, acc, isem, gsem, ssem):
        c = lax.axis_index("c")
        s = lax.axis_index("s")
        wid = s * _NC + c
        ebase = wid * ept

        def zrow(k, _):
            def zd(j, _):
                gbuf[0, k, pl.ds(j * _NL, _NL)] = jnp.zeros((_NL,), jnp.float32)
                return 0
            return lax.fori_loop(0, D // _NL, zd, 0)
        lax.fori_loop(0, K, zrow, 0)
        rbase = s * rpt
        nfull, tail = divmod(rpt, K)
        for t in range(nfull):
            pltpu.sync_copy(gbuf.at[0], acc.at[pl.ds(rbase + t * K, K)])
        if tail:
            pltpu.sync_copy(gbuf.at[0].at[pl.ds(0, tail)],
                            acc.at[pl.ds(rbase + nfull * K, tail)])

        @pl.when(s == 0)
        def _zero_rem():
            pltpu.sync_copy(gbuf.at[0].at[pl.ds(0, rem)],
                            acc.at[pl.ds(_NS * rpt, rem)])
        plsc.subcore_barrier()

        def istart(t, slot):
            off = ebase + t * K
            pltpu.async_copy(row_hbm.at[pl.ds(off, K)], rowv.at[slot], isem)
            pltpu.async_copy(col_hbm.at[pl.ds(off, K)], colv.at[slot], isem)
            pltpu.async_copy(w_hbm.at[pl.ds(off, K)], wv.at[slot], isem)

        def iwait():
            for r in (rowv, colv, wv):
                pltpu.make_async_copy(row_hbm.at[pl.ds(0, K)], r.at[0], isem).wait()

        def gstart(islot, b):
            pltpu.async_copy(mat_hbm.at[colv.at[islot]], gbuf.at[b], gsem)

        def gwait():
            pltpu.make_async_copy(mat_hbm.at[colv.at[0]], gbuf.at[0], gsem).wait()

        def sstart(islot, b):
            pltpu.async_copy(gbuf.at[b], acc.at[rowv.at[islot]], ssem, add=True)

        def swait():
            pltpu.make_async_copy(gbuf.at[0], acc.at[rowv.at[0]], ssem).wait()

        def scale(islot, b):
            def sg(g, _):
                w16 = wv[islot, pl.ds(g * _NL, _NL)]
                for k in range(_NL):
                    w = w16[k]
                    e = g * _NL + k
                    for j in range(d_scale // _NL):
                        sl = pl.ds(j * _NL, _NL)
                        gbuf[b, e, sl] = gbuf[b, e, sl] * w
                return 0
            lax.fori_loop(0, K // _NL, sg, 0)

        def step(t, j):
            b = j % NB
            gwait()
            scale(j % NBI, b)
            sstart(j % NBI, b)

            @pl.when(t >= 2)
            def _():
                swait()

            @pl.when(t <= nchunk - 3)
            def _():
                iwait()
                gstart((j + 2) % NBI, (b + 2) % NB)

            @pl.when(t <= nchunk - 5)
            def _():
                istart(t + 4, (j + 4) % NBI)

        for t in range(4):
            istart(t, t)
        iwait()
        iwait()
        gstart(0, 0)
        gstart(1, 1)

        def group(g, _):
            for j in range(NU):
                step(g * NU + j, j)
            return 0
        lax.fori_loop(0, nchunk // NU, group, 0)
        for t in range(NU * (nchunk // NU), nchunk):
            step(t, t % NU)
        swait()
        swait()
        plsc.subcore_barrier()

        pltpu.sync_copy(acc.at[pl.ds(rbase, rpt)], out_hbm.at[c, pl.ds(rbase, rpt)])

        @pl.when(s == 0)
        def _write_rem():
            pltpu.sync_copy(acc.at[pl.ds(_NS * rpt, rem)],
                            out_hbm.at[c, pl.ds(_NS * rpt, rem)])

    return spmm


_spmm128 = _make_spmm(128)
_spmm48 = _make_spmm(128, d_scale=48)


def _tc_matmul1(x, W1):
    bn = 1000

    def body(x_ref, w_ref, o_ref):
        o_ref[...] = jnp.dot(x_ref[...], w_ref[...],
                             preferred_element_type=jnp.float32)

    return pl.pallas_call(
        body,
        grid=(_N // bn,),
        in_specs=[pl.BlockSpec((bn, 128), lambda i: (i, 0)),
                  pl.BlockSpec((128, 128), lambda i: (0, 0))],
        out_specs=pl.BlockSpec((bn, 128), lambda i: (i, 0)),
        out_shape=jax.ShapeDtypeStruct((_N, 128), jnp.float32),
    )(x, W1)


def _tc_layer2(p0, p1, b1, W2p):
    bn = 1000

    def body(p0_ref, p1_ref, b_ref, w_ref, o_ref):
        h = jnp.maximum(p0_ref[...] + p1_ref[...] + b_ref[...], 0.0)
        o_ref[...] = jnp.dot(h, w_ref[...], preferred_element_type=jnp.float32)

    return pl.pallas_call(
        body,
        grid=(_N // bn,),
        in_specs=[pl.BlockSpec((bn, 128), lambda i: (i, 0)),
                  pl.BlockSpec((bn, 128), lambda i: (i, 0)),
                  pl.BlockSpec((1, 128), lambda i: (0, 0)),
                  pl.BlockSpec((128, 128), lambda i: (0, 0))],
        out_specs=pl.BlockSpec((bn, 128), lambda i: (i, 0)),
        out_shape=jax.ShapeDtypeStruct((_N, 128), jnp.float32),
    )(p0, p1, b1, W2p)


def _tc_final(q0, q1, b2):
    bn = 1000

    def body(q0_ref, q1_ref, b_ref, o_ref):
        z = (q0_ref[...] + q1_ref[...])[:, :40] + b_ref[...]
        z = z - jnp.max(z, axis=1, keepdims=True)
        o_ref[...] = z - jnp.log(jnp.sum(jnp.exp(z), axis=1, keepdims=True))

    return pl.pallas_call(
        body,
        grid=(_N // bn,),
        in_specs=[pl.BlockSpec((bn, 128), lambda i: (i, 0)),
                  pl.BlockSpec((bn, 128), lambda i: (i, 0)),
                  pl.BlockSpec((1, 40), lambda i: (0, 0))],
        out_specs=pl.BlockSpec((bn, 40), lambda i: (i, 0)),
        out_shape=jax.ShapeDtypeStruct((_N, 40), jnp.float32),
    )(q0, q1, b2)


def kernel(x, edge_index, edge_weight, W1, b1, W2, b2):
    row = edge_index[0].astype(jnp.int32)
    col = edge_index[1].astype(jnp.int32)
    ew = edge_weight.astype(jnp.float32)

    s1 = _tc_matmul1(x, W1)
    p = _spmm128(row, col, ew, s1)
    W2p = jnp.pad(W2, ((0, 0), (0, 128 - W2.shape[1])))
    s2 = _tc_layer2(p[0], p[1], b1.reshape(1, -1), W2p)
    q = _spmm48(row, col, ew, s2)
    return _tc_final(q[0], q[1], b2.reshape(1, -1))
